# baseline (speedup 1.0000x reference)
import functools

import jax
import jax.numpy as jnp
from jax import lax
from jax.experimental import pallas as pl
from jax.experimental.pallas import tpu as pltpu
from jax.experimental.pallas import tpu_sc as plsc

_NC = 2
_NS = 16
_NW = _NC * _NS


def _edge_plan(n, e, ch):
    ept = e // _NW
    nbuf = 2
    ngroups = ept // ch // nbuf
    npre = ngroups * nbuf * ch
    tail = ept - npre
    rpt = -(-(n // _NS) // 8) * 8
    return ept, nbuf, ngroups, npre, tail, rpt, rpt * _NS


def _sc_aggregate(row, col, feat, zacc, n, e, d, ch):
    ept, nbuf, ngroups, npre, tail, rpt, n_pad = _edge_plan(n, e, ch)
    mesh = plsc.VectorSubcoreMesh(
        core_axis_name="c", subcore_axis_name="s",
        num_cores=_NC, num_subcores=_NS)

    @functools.partial(
        pl.kernel,
        out_type=jax.ShapeDtypeStruct((_NC, n_pad, d), jnp.float32),
        mesh=mesh,
        scratch_types=[
            pltpu.VMEM((ept,), jnp.int32),
            [pltpu.VMEM((ch,), jnp.int32) for _ in range(nbuf)],
            [pltpu.VMEM((ch, d), jnp.float32) for _ in range(nbuf)],
            pltpu.VMEM((tail,), jnp.int32),
            pltpu.VMEM_SHARED((n_pad, d), jnp.float32),
            [pltpu.SemaphoreType.DMA for _ in range(nbuf)],
            [pltpu.SemaphoreType.DMA for _ in range(nbuf)],
            pltpu.SemaphoreType.DMA,
        ],
    )
    def sc_kernel(row_h, col_h, feat_h, zacc_h, acc_out,
                  row_all, col_bufs, rows_bufs, tcol,
                  acc_sh, sems_c, sems_g, sem_b):
        c = lax.axis_index("c")
        s = lax.axis_index("s")
        wid = s * _NC + c
        base = wid * ept
        pltpu.async_copy(row_h.at[pl.ds(base, ept)], row_all, sem_b)
        r0 = s * rpt
        pltpu.sync_copy(zacc_h, acc_sh.at[pl.ds(r0, rpt)])
        pltpu.make_async_copy(row_h.at[pl.ds(base, ept)], row_all, sem_b).wait()
        plsc.subcore_barrier()

        def issue(j, b):
            pltpu.async_copy(col_h.at[pl.ds(base + j * ch, ch)],
                             col_bufs[b], sems_c[b])
            pltpu.async_copy(feat_h.at[row_all.at[pl.ds(j * ch, ch)]],
                             rows_bufs[b], sems_g[b])

        def drain(j, b):
            pltpu.make_async_copy(col_h.at[pl.ds(base + j * ch, ch)],
                                  col_bufs[b], sems_c[b]).wait()
            pltpu.make_async_copy(feat_h.at[row_all.at[pl.ds(j * ch, ch)]],
                                  rows_bufs[b], sems_g[b]).wait()

        for b in range(nbuf):
            issue(b, b)

        def group(g, carry):
            for b in range(nbuf):
                j = g * nbuf + b
                drain(j, b)
                pltpu.sync_copy(rows_bufs[b], acc_sh.at[col_bufs[b]], add=True)
                jn = j + nbuf

                @pl.when(jn * ch < npre)
                def _():
                    issue(jn, b)
            return carry

        lax.fori_loop(0, ngroups, group, 0)

        if tail:
            trows = rows_bufs[0].at[pl.ds(0, tail)]
            pltpu.async_copy(col_h.at[pl.ds(base + npre, tail)], tcol, sem_b)
            pltpu.async_copy(feat_h.at[row_all.at[pl.ds(npre, tail)]],
                             trows, sems_g[0])
            pltpu.make_async_copy(col_h.at[pl.ds(base + npre, tail)],
                                  tcol, sem_b).wait()
            pltpu.make_async_copy(feat_h.at[row_all.at[pl.ds(npre, tail)]],
                                  trows, sems_g[0]).wait()
            pltpu.sync_copy(trows, acc_sh.at[tcol], add=True)

        plsc.subcore_barrier()
        pltpu.sync_copy(acc_sh.at[pl.ds(r0, rpt)],
                        acc_out.at[c, pl.ds(r0, rpt)])

    return sc_kernel(row, col, feat, zacc)


def _sc_degree(col, zacc, ones, n, e, d, ch):
    ept, nbuf, ngroups, npre, tail, rpt, n_pad = _edge_plan(n, e, ch)
    mesh = plsc.VectorSubcoreMesh(
        core_axis_name="c", subcore_axis_name="s",
        num_cores=_NC, num_subcores=_NS)

    @functools.partial(
        pl.kernel,
        out_type=jax.ShapeDtypeStruct((_NC, n_pad, d), jnp.float32),
        mesh=mesh,
        scratch_types=[
            [pltpu.VMEM((ch,), jnp.int32) for _ in range(nbuf)],
            pltpu.VMEM((ch, d), jnp.float32),
            pltpu.VMEM((tail,), jnp.int32),
            pltpu.VMEM_SHARED((n_pad, d), jnp.float32),
            [pltpu.SemaphoreType.DMA for _ in range(nbuf)],
        ],
    )
    def sc_kernel(col_h, zacc_h, ones_h, deg_out,
                  col_bufs, ones_v, tcol, deg_sh, sems_c):
        c = lax.axis_index("c")
        s = lax.axis_index("s")
        wid = s * _NC + c
        base = wid * ept
        r0 = s * rpt
        pltpu.sync_copy(zacc_h, deg_sh.at[pl.ds(r0, rpt)])
        pltpu.sync_copy(ones_h, ones_v)
        plsc.subcore_barrier()

        for b in range(nbuf):
            pltpu.async_copy(col_h.at[pl.ds(base + b * ch, ch)],
                             col_bufs[b], sems_c[b])

        def group(g, carry):
            for b in range(nbuf):
                j = g * nbuf + b
                pltpu.make_async_copy(col_h.at[pl.ds(base + j * ch, ch)],
                                      col_bufs[b], sems_c[b]).wait()
                pltpu.sync_copy(ones_v, deg_sh.at[col_bufs[b]], add=True)
                jn = j + nbuf

                @pl.when(jn * ch < npre)
                def _():
                    pltpu.async_copy(col_h.at[pl.ds(base + jn * ch, ch)],
                                     col_bufs[b], sems_c[b])
            return carry

        lax.fori_loop(0, ngroups, group, 0)

        if tail:
            pltpu.sync_copy(col_h.at[pl.ds(base + npre, tail)], tcol)
            pltpu.sync_copy(ones_v.at[pl.ds(0, tail)], deg_sh.at[tcol],
                            add=True)

        plsc.subcore_barrier()
        pltpu.sync_copy(deg_sh.at[pl.ds(r0, rpt)],
                        deg_out.at[c, pl.ds(r0, rpt)])

    return sc_kernel(col, zacc, ones)


def _final_tc(feat, wst, wnt, b2, acc, deg2, n, d, r):
    def body(x_ref, wst_ref, wnt_ref, b_ref, a0_ref, a1_ref, d0_ref, d1_ref,
             o_ref):
        x = x_ref[...]
        a = a0_ref[0] + a1_ref[0]
        deg = d0_ref[0, :, 0:1] + d1_ref[0, :, 0:1]
        dinv = jnp.where(deg > 0, 1.0 / deg, 0.0)
        o_ref[...] = (
            jnp.dot(x, wst_ref[...], preferred_element_type=jnp.float32)
            + jnp.dot(a * dinv, wnt_ref[...],
                      preferred_element_type=jnp.float32)
            + b_ref[...])

    return pl.pallas_call(
        body,
        grid=(n // r,),
        in_specs=[
            pl.BlockSpec((r, d), lambda i: (i, 0)),
            pl.BlockSpec((d, d), lambda i: (0, 0)),
            pl.BlockSpec((d, d), lambda i: (0, 0)),
            pl.BlockSpec((1, d), lambda i: (0, 0)),
            pl.BlockSpec((1, r, d), lambda i: (0, i, 0)),
            pl.BlockSpec((1, r, d), lambda i: (1, i, 0)),
            pl.BlockSpec((1, r, d), lambda i: (0, i, 0)),
            pl.BlockSpec((1, r, d), lambda i: (1, i, 0)),
        ],
        out_specs=pl.BlockSpec((r, d), lambda i: (i, 0)),
        out_shape=jax.ShapeDtypeStruct((n, d), jnp.float32),
    )(feat, wst, wnt, b2, acc, acc, deg2, deg2)


def kernel(feat, edge_index, W_neigh, W_self, b_self):
    n, d = feat.shape
    e = edge_index.shape[1]
    row = edge_index[0]
    col = edge_index[1]
    ch_a, ch_g = 128, 128
    rpt = _edge_plan(n, e, ch_a)[5]
    zacc = jnp.zeros((rpt, d), jnp.float32)
    ones = jnp.ones((ch_g, d), jnp.float32)
    acc = _sc_aggregate(row, col, feat, zacc, n, e, d, ch_a)
    deg2 = _sc_degree(col, zacc, ones, n, e, d, ch_g)
    return _final_tc(feat, W_self.T, W_neigh.T, b_self.reshape(1, d),
                     acc, deg2, n, d, 2000)


# speedup vs baseline: 1.0284x; 1.0284x over previous
import functools

import jax
import jax.numpy as jnp
from jax import lax
from jax.experimental import pallas as pl
from jax.experimental.pallas import tpu as pltpu
from jax.experimental.pallas import tpu_sc as plsc

_NC = 2
_NS = 16
_NW = _NC * _NS


def _edge_plan(n, e, ch, nbuf=2):
    ept = e // _NW
    ngroups = ept // ch // nbuf
    npre = ngroups * nbuf * ch
    tail = ept - npre
    rpt = -(-(n // _NS) // 8) * 8
    return ept, nbuf, ngroups, npre, tail, rpt, rpt * _NS


def _sc_aggregate(row, col, feat, zacc, n, e, d, ch, nbuf):
    ept, nbuf, ngroups, npre, tail, rpt, n_pad = _edge_plan(n, e, ch, nbuf)
    mesh = plsc.VectorSubcoreMesh(
        core_axis_name="c", subcore_axis_name="s",
        num_cores=_NC, num_subcores=_NS)

    @functools.partial(
        pl.kernel,
        out_type=jax.ShapeDtypeStruct((_NC, n_pad, d), jnp.float32),
        mesh=mesh,
        scratch_types=[
            pltpu.VMEM((ept,), jnp.int32),
            [pltpu.VMEM((ch,), jnp.int32) for _ in range(nbuf)],
            [pltpu.VMEM((ch, d), jnp.float32) for _ in range(nbuf)],
            pltpu.VMEM((tail,), jnp.int32),
            pltpu.VMEM_SHARED((n_pad, d), jnp.float32),
            [pltpu.SemaphoreType.DMA for _ in range(nbuf)],
            [pltpu.SemaphoreType.DMA for _ in range(nbuf)],
            pltpu.SemaphoreType.DMA,
        ],
    )
    def sc_kernel(row_h, col_h, feat_h, zacc_h, acc_out,
                  row_all, col_bufs, rows_bufs, tcol,
                  acc_sh, sems_c, sems_g, sem_b):
        c = lax.axis_index("c")
        s = lax.axis_index("s")
        wid = s * _NC + c
        base = wid * ept
        pltpu.async_copy(row_h.at[pl.ds(base, ept)], row_all, sem_b)
        r0 = s * rpt
        pltpu.sync_copy(zacc_h, acc_sh.at[pl.ds(r0, rpt)])
        pltpu.make_async_copy(row_h.at[pl.ds(base, ept)], row_all, sem_b).wait()
        plsc.subcore_barrier()

        def issue(j, b):
            pltpu.async_copy(col_h.at[pl.ds(base + j * ch, ch)],
                             col_bufs[b], sems_c[b])
            pltpu.async_copy(feat_h.at[row_all.at[pl.ds(j * ch, ch)]],
                             rows_bufs[b], sems_g[b])

        def drain(j, b):
            pltpu.make_async_copy(col_h.at[pl.ds(base + j * ch, ch)],
                                  col_bufs[b], sems_c[b]).wait()
            pltpu.make_async_copy(feat_h.at[row_all.at[pl.ds(j * ch, ch)]],
                                  rows_bufs[b], sems_g[b]).wait()

        for b in range(nbuf):
            issue(b, b)

        def group(g, carry):
            for b in range(nbuf):
                j = g * nbuf + b
                drain(j, b)
                pltpu.sync_copy(rows_bufs[b], acc_sh.at[col_bufs[b]], add=True)
                jn = j + nbuf

                @pl.when(jn * ch < npre)
                def _():
                    issue(jn, b)
            return carry

        lax.fori_loop(0, ngroups, group, 0)

        if tail:
            trows = rows_bufs[0].at[pl.ds(0, tail)]
            pltpu.async_copy(col_h.at[pl.ds(base + npre, tail)], tcol, sem_b)
            pltpu.async_copy(feat_h.at[row_all.at[pl.ds(npre, tail)]],
                             trows, sems_g[0])
            pltpu.make_async_copy(col_h.at[pl.ds(base + npre, tail)],
                                  tcol, sem_b).wait()
            pltpu.make_async_copy(feat_h.at[row_all.at[pl.ds(npre, tail)]],
                                  trows, sems_g[0]).wait()
            pltpu.sync_copy(trows, acc_sh.at[tcol], add=True)

        plsc.subcore_barrier()
        pltpu.sync_copy(acc_sh.at[pl.ds(r0, rpt)],
                        acc_out.at[c, pl.ds(r0, rpt)])

    return sc_kernel(row, col, feat, zacc)


def _sc_degree(col, zacc, ones, n, e, d, ch):
    ept, nbuf, ngroups, npre, tail, rpt, n_pad = _edge_plan(n, e, ch)
    mesh = plsc.VectorSubcoreMesh(
        core_axis_name="c", subcore_axis_name="s",
        num_cores=_NC, num_subcores=_NS)

    @functools.partial(
        pl.kernel,
        out_type=jax.ShapeDtypeStruct((_NC, n_pad, d), jnp.float32),
        mesh=mesh,
        scratch_types=[
            [pltpu.VMEM((ch,), jnp.int32) for _ in range(nbuf)],
            pltpu.VMEM((ch, d), jnp.float32),
            pltpu.VMEM((tail,), jnp.int32),
            pltpu.VMEM_SHARED((n_pad, d), jnp.float32),
            [pltpu.SemaphoreType.DMA for _ in range(nbuf)],
        ],
    )
    def sc_kernel(col_h, zacc_h, ones_h, deg_out,
                  col_bufs, ones_v, tcol, deg_sh, sems_c):
        c = lax.axis_index("c")
        s = lax.axis_index("s")
        wid = s * _NC + c
        base = wid * ept
        r0 = s * rpt
        pltpu.sync_copy(zacc_h, deg_sh.at[pl.ds(r0, rpt)])
        pltpu.sync_copy(ones_h, ones_v)
        plsc.subcore_barrier()

        for b in range(nbuf):
            pltpu.async_copy(col_h.at[pl.ds(base + b * ch, ch)],
                             col_bufs[b], sems_c[b])

        def group(g, carry):
            for b in range(nbuf):
                j = g * nbuf + b
                pltpu.make_async_copy(col_h.at[pl.ds(base + j * ch, ch)],
                                      col_bufs[b], sems_c[b]).wait()
                pltpu.sync_copy(ones_v, deg_sh.at[col_bufs[b]], add=True)
                jn = j + nbuf

                @pl.when(jn * ch < npre)
                def _():
                    pltpu.async_copy(col_h.at[pl.ds(base + jn * ch, ch)],
                                     col_bufs[b], sems_c[b])
            return carry

        lax.fori_loop(0, ngroups, group, 0)

        if tail:
            pltpu.sync_copy(col_h.at[pl.ds(base + npre, tail)], tcol)
            pltpu.sync_copy(ones_v.at[pl.ds(0, tail)], deg_sh.at[tcol],
                            add=True)

        plsc.subcore_barrier()
        pltpu.sync_copy(deg_sh.at[pl.ds(r0, rpt)],
                        deg_out.at[c, pl.ds(r0, rpt)])

    return sc_kernel(col, zacc, ones)


def _final_tc(feat, wst, wnt, b2, acc, deg2, n, d, r):
    def body(x_ref, wst_ref, wnt_ref, b_ref, a0_ref, a1_ref, d0_ref, d1_ref,
             o_ref):
        x = x_ref[...]
        a = a0_ref[0] + a1_ref[0]
        deg = d0_ref[0, :, 0:1] + d1_ref[0, :, 0:1]
        dinv = jnp.where(deg > 0, 1.0 / deg, 0.0)
        o_ref[...] = (
            jnp.dot(x, wst_ref[...], preferred_element_type=jnp.float32)
            + jnp.dot(a * dinv, wnt_ref[...],
                      preferred_element_type=jnp.float32)
            + b_ref[...])

    return pl.pallas_call(
        body,
        grid=(n // r,),
        in_specs=[
            pl.BlockSpec((r, d), lambda i: (i, 0)),
            pl.BlockSpec((d, d), lambda i: (0, 0)),
            pl.BlockSpec((d, d), lambda i: (0, 0)),
            pl.BlockSpec((1, d), lambda i: (0, 0)),
            pl.BlockSpec((1, r, d), lambda i: (0, i, 0)),
            pl.BlockSpec((1, r, d), lambda i: (1, i, 0)),
            pl.BlockSpec((1, r, d), lambda i: (0, i, 0)),
            pl.BlockSpec((1, r, d), lambda i: (1, i, 0)),
        ],
        out_specs=pl.BlockSpec((r, d), lambda i: (i, 0)),
        out_shape=jax.ShapeDtypeStruct((n, d), jnp.float32),
    )(feat, wst, wnt, b2, acc, acc, deg2, deg2)


def kernel(feat, edge_index, W_neigh, W_self, b_self):
    n, d = feat.shape
    e = edge_index.shape[1]
    row = edge_index[0]
    col = edge_index[1]
    ch_a, ch_g = 64, 128
    rpt = _edge_plan(n, e, ch_a)[5]
    zacc = jnp.zeros((rpt, d), jnp.float32)
    ones = jnp.ones((ch_g, d), jnp.float32)
    acc = _sc_aggregate(row, col, feat, zacc, n, e, d, ch_a, 3)
    deg2 = _sc_degree(col, zacc, ones, n, e, d, ch_g)
    return _final_tc(feat, W_self.T, W_neigh.T, b_self.reshape(1, d),
                     acc, deg2, n, d, 2000)
